# baseline (speedup 1.0000x reference)
import functools

import jax
import jax.numpy as jnp
from jax import lax
from jax.experimental import pallas as pl
from jax.experimental.pallas import tpu as pltpu
from jax.experimental.pallas import tpu_sc as plsc

N = 10000
E = 160000
LB = 128
CW = 256
NS = 16
CPT = 40
E_PAD = NS * CPT * CW
ACC_ROWS = N + 8
RPT = 624
TAIL = N - NS * RPT

_f32 = jnp.float32


def _make_seg_sum(cb_total):
    my = cb_total // 2
    mesh = plsc.VectorSubcoreMesh(core_axis_name="c", subcore_axis_name="s")

    def body(*refs):
        src1, dst1, zs = refs[0:3]
        h_refs = refs[3:3 + cb_total]
        out_refs = refs[3 + cb_total:3 + 2 * cb_total]
        rest = refs[3 + 2 * cb_total:]
        shared, sidx1, didx1, rows = rest[0:4]
        gsem, ssem = rest[4:6]
        c = lax.axis_index("c")
        s = lax.axis_index("s")
        rbase = s * RPT
        for cb in range(cb_total):
            @pl.when(c == cb // my)
            def _(cb=cb):
                pltpu.sync_copy(zs.at[pl.ds(0, RPT)],
                                shared.at[pl.ds(rbase, RPT)])

                @pl.when(s == 0)
                def _():
                    pltpu.sync_copy(
                        zs.at[pl.ds(0, ACC_ROWS - NS * RPT)],
                        shared.at[pl.ds(NS * RPT, ACC_ROWS - NS * RPT)])
                plsc.subcore_barrier()

                for half in range(2):
                    ebase = (s * CPT + half * (CPT // 2)) * CW
                    pltpu.sync_copy(src1.at[pl.ds(ebase, CPT // 2 * CW)],
                                    sidx1)
                    pltpu.sync_copy(dst1.at[pl.ds(ebase, CPT // 2 * CW)],
                                    didx1)

                    def chunk(j, carry):
                        off = pl.multiple_of(j * CW, CW)
                        pltpu.async_copy(
                            h_refs[cb].at[sidx1.at[pl.ds(off, CW)]],
                            rows, gsem).wait()
                        pltpu.async_copy(
                            rows, shared.at[didx1.at[pl.ds(off, CW)]], ssem,
                            add=True).wait()
                        return carry

                    lax.fori_loop(0, CPT // 2, chunk, 0)

                plsc.subcore_barrier()
                pltpu.sync_copy(shared.at[pl.ds(rbase, RPT)],
                                out_refs[cb].at[pl.ds(rbase, RPT)])

                @pl.when(s == 0)
                def _():
                    pltpu.sync_copy(shared.at[pl.ds(NS * RPT, TAIL)],
                                    out_refs[cb].at[pl.ds(NS * RPT, TAIL)])

    out_type = tuple(jax.ShapeDtypeStruct((N, LB), _f32)
                     for _ in range(cb_total))
    scratch = (
        [pltpu.VMEM_SHARED((ACC_ROWS, LB), _f32),
         pltpu.VMEM((CPT // 2 * CW,), jnp.int32),
         pltpu.VMEM((CPT // 2 * CW,), jnp.int32),
         pltpu.VMEM((CW, LB), _f32)] +
        [pltpu.SemaphoreType.DMA] * 2)
    return pl.kernel(body, out_type=out_type, mesh=mesh,
                     scratch_types=scratch)


_ROWS = 1000


def _mlp_layer(cbi):
    ci = cbi * LB
    nout = 512 // LB

    def body(*refs):
        h_refs = refs[0:cbi]
        a_refs = refs[cbi:2 * cbi]
        w1, b1, w2, b2 = refs[2 * cbi:2 * cbi + 4]
        o_refs = refs[2 * cbi + 4:]
        z = jnp.concatenate(
            [h_refs[i][...] + a_refs[i][...] for i in range(cbi)], axis=1)
        t = jnp.maximum(
            jnp.dot(z, w1[...], preferred_element_type=_f32) + b1[...], 0.0)
        y = jnp.maximum(
            jnp.dot(t, w2[...], preferred_element_type=_f32) + b2[...], 0.0)
        for i in range(nout):
            o_refs[i][...] = y[:, LB * i:LB * (i + 1)]

    blk = pl.BlockSpec((_ROWS, LB), lambda i: (i, 0))
    in_specs = (
        [blk] * cbi + [blk] * cbi +
        [pl.BlockSpec((ci, 512), lambda i: (0, 0)),
         pl.BlockSpec((1, 512), lambda i: (0, 0)),
         pl.BlockSpec((512, 512), lambda i: (0, 0)),
         pl.BlockSpec((1, 512), lambda i: (0, 0))])
    out_specs = [blk] * nout
    return pl.pallas_call(
        body,
        grid=(N // _ROWS,),
        in_specs=in_specs,
        out_specs=out_specs,
        out_shape=tuple(jax.ShapeDtypeStruct((N, LB), _f32)
                        for _ in range(nout)),
    )


def _mlp_final():
    nin = 512 // LB

    def body(*refs):
        h_refs = refs[0:nin]
        a_refs = refs[nin:2 * nin]
        w1, b1, w2, b2, lw, lb_, o_ref = refs[2 * nin:]
        z = jnp.concatenate(
            [h_refs[i][...] + a_refs[i][...] for i in range(nin)], axis=1)
        t = jnp.maximum(
            jnp.dot(z, w1[...], preferred_element_type=_f32) + b1[...], 0.0)
        y = jnp.maximum(
            jnp.dot(t, w2[...], preferred_element_type=_f32) + b2[...], 0.0)
        o_ref[...] = jnp.dot(y, lw[...], preferred_element_type=_f32) + lb_[...]

    blk = pl.BlockSpec((_ROWS, LB), lambda i: (i, 0))
    in_specs = (
        [blk] * (2 * nin) +
        [pl.BlockSpec((512, 512), lambda i: (0, 0)),
         pl.BlockSpec((1, 512), lambda i: (0, 0)),
         pl.BlockSpec((512, 512), lambda i: (0, 0)),
         pl.BlockSpec((1, 512), lambda i: (0, 0)),
         pl.BlockSpec((512, 256), lambda i: (0, 0)),
         pl.BlockSpec((1, 256), lambda i: (0, 0))])
    return pl.pallas_call(
        body,
        grid=(N // _ROWS,),
        in_specs=in_specs,
        out_specs=pl.BlockSpec((_ROWS, 256), lambda i: (i, 0)),
        out_shape=jax.ShapeDtypeStruct((N, 256), _f32),
    )


def kernel(x, edge_index, W1_0, b1_0, W2_0, b2_0, W1_1, b1_1, W2_1, b2_1,
           W1_2, b1_2, W2_2, b2_2, lin_W, lin_b):
    src = edge_index[0]
    dst = edge_index[1]
    pad = E_PAD - E
    src = jnp.concatenate([src, jnp.zeros((pad,), jnp.int32)])
    dst = jnp.concatenate([dst, jnp.full((pad,), N, jnp.int32)])
    zeros_rows = jnp.zeros((RPT, LB), _f32)

    nb_in = 256 // LB
    nb_hid = 512 // LB
    seg_in = _make_seg_sum(nb_in)
    seg_hid = _make_seg_sum(nb_hid)
    mlp_in = _mlp_layer(nb_in)
    mlp_hid = _mlp_layer(nb_hid)
    mlpf = _mlp_final()

    h = [x[:, LB * i:LB * (i + 1)] for i in range(nb_in)]
    agg = seg_in(src, dst, zeros_rows, *h)
    h = mlp_in(*h, *agg, W1_0, b1_0.reshape(1, -1), W2_0, b2_0.reshape(1, -1))
    agg = seg_hid(src, dst, zeros_rows, *h)
    h = mlp_hid(*h, *agg, W1_1, b1_1.reshape(1, -1), W2_1, b2_1.reshape(1, -1))
    agg = seg_hid(src, dst, zeros_rows, *h)
    return mlpf(*h, *agg, W1_2, b1_2.reshape(1, -1), W2_2, b2_2.reshape(1, -1),
                lin_W, lin_b.reshape(1, -1))


# speedup vs baseline: 1.0659x; 1.0659x over previous
import functools

import jax
import jax.numpy as jnp
from jax import lax
from jax.experimental import pallas as pl
from jax.experimental.pallas import tpu as pltpu
from jax.experimental.pallas import tpu_sc as plsc

N = 10000
E = 160000
LB = 128
CW = 128
NS = 16
CPT = 80
E_PAD = NS * CPT * CW
ACC_ROWS = N + 8
RPT = 624
TAIL = N - NS * RPT

_f32 = jnp.float32


def _make_seg_sum(cb_total):
    my = cb_total // 2
    mesh = plsc.VectorSubcoreMesh(core_axis_name="c", subcore_axis_name="s")

    def body(*refs):
        src1, dst1, zs = refs[0:3]
        h_refs = refs[3:3 + cb_total]
        out_refs = refs[3 + cb_total:3 + 2 * cb_total]
        rest = refs[3 + 2 * cb_total:]
        shared, sidx1, didx1, r0, r1 = rest[0:5]
        rows = (r0, r1)
        gsem = rest[5:7]
        ssem = rest[7:9]
        c = lax.axis_index("c")
        s = lax.axis_index("s")
        rbase = s * RPT
        for cb in range(cb_total):
            @pl.when(c == cb // my)
            def _(cb=cb):
                pltpu.sync_copy(zs.at[pl.ds(0, RPT)],
                                shared.at[pl.ds(rbase, RPT)])

                @pl.when(s == 0)
                def _():
                    pltpu.sync_copy(
                        zs.at[pl.ds(0, ACC_ROWS - NS * RPT)],
                        shared.at[pl.ds(NS * RPT, ACC_ROWS - NS * RPT)])
                plsc.subcore_barrier()

                for half in range(2):
                    ebase = (s * CPT + half * (CPT // 2)) * CW
                    pltpu.sync_copy(src1.at[pl.ds(ebase, CPT // 2 * CW)],
                                    sidx1)
                    pltpu.sync_copy(dst1.at[pl.ds(ebase, CPT // 2 * CW)],
                                    didx1)

                    def pair(g, carry):
                        for p in range(2):
                            j = g * 2 + p

                            @pl.when(j >= 2)
                            def _():
                                pltpu.make_async_copy(
                                    zs.at[pl.ds(0, CW)], rows[p],
                                    ssem[p]).wait()

                            off = pl.multiple_of(j * CW, CW)
                            pltpu.async_copy(
                                h_refs[cb].at[sidx1.at[pl.ds(off, CW)]],
                                rows[p], gsem[p]).wait()
                            pltpu.async_copy(
                                rows[p],
                                shared.at[didx1.at[pl.ds(off, CW)]],
                                ssem[p], add=True)
                        return carry

                    lax.fori_loop(0, CPT // 4, pair, 0)
                    for p in range(2):
                        pltpu.make_async_copy(zs.at[pl.ds(0, CW)], rows[p],
                                              ssem[p]).wait()

                plsc.subcore_barrier()
                pltpu.sync_copy(shared.at[pl.ds(rbase, RPT)],
                                out_refs[cb].at[pl.ds(rbase, RPT)])

                @pl.when(s == 0)
                def _():
                    pltpu.sync_copy(shared.at[pl.ds(NS * RPT, TAIL)],
                                    out_refs[cb].at[pl.ds(NS * RPT, TAIL)])

    out_type = tuple(jax.ShapeDtypeStruct((N, LB), _f32)
                     for _ in range(cb_total))
    scratch = (
        [pltpu.VMEM_SHARED((ACC_ROWS, LB), _f32),
         pltpu.VMEM((CPT // 2 * CW,), jnp.int32),
         pltpu.VMEM((CPT // 2 * CW,), jnp.int32),
         pltpu.VMEM((CW, LB), _f32),
         pltpu.VMEM((CW, LB), _f32)] +
        [pltpu.SemaphoreType.DMA] * 4)
    return pl.kernel(body, out_type=out_type, mesh=mesh,
                     scratch_types=scratch)


_ROWS = 1000


def _mlp_layer(cbi):
    ci = cbi * LB
    nout = 512 // LB

    def body(*refs):
        h_refs = refs[0:cbi]
        a_refs = refs[cbi:2 * cbi]
        w1, b1, w2, b2 = refs[2 * cbi:2 * cbi + 4]
        o_refs = refs[2 * cbi + 4:]
        z = jnp.concatenate(
            [h_refs[i][...] + a_refs[i][...] for i in range(cbi)], axis=1)
        t = jnp.maximum(
            jnp.dot(z, w1[...], preferred_element_type=_f32) + b1[...], 0.0)
        y = jnp.maximum(
            jnp.dot(t, w2[...], preferred_element_type=_f32) + b2[...], 0.0)
        for i in range(nout):
            o_refs[i][...] = y[:, LB * i:LB * (i + 1)]

    blk = pl.BlockSpec((_ROWS, LB), lambda i: (i, 0))
    in_specs = (
        [blk] * cbi + [blk] * cbi +
        [pl.BlockSpec((ci, 512), lambda i: (0, 0)),
         pl.BlockSpec((1, 512), lambda i: (0, 0)),
         pl.BlockSpec((512, 512), lambda i: (0, 0)),
         pl.BlockSpec((1, 512), lambda i: (0, 0))])
    out_specs = [blk] * nout
    return pl.pallas_call(
        body,
        grid=(N // _ROWS,),
        in_specs=in_specs,
        out_specs=out_specs,
        out_shape=tuple(jax.ShapeDtypeStruct((N, LB), _f32)
                        for _ in range(nout)),
    )


def _mlp_final():
    nin = 512 // LB

    def body(*refs):
        h_refs = refs[0:nin]
        a_refs = refs[nin:2 * nin]
        w1, b1, w2, b2, lw, lb_, o_ref = refs[2 * nin:]
        z = jnp.concatenate(
            [h_refs[i][...] + a_refs[i][...] for i in range(nin)], axis=1)
        t = jnp.maximum(
            jnp.dot(z, w1[...], preferred_element_type=_f32) + b1[...], 0.0)
        y = jnp.maximum(
            jnp.dot(t, w2[...], preferred_element_type=_f32) + b2[...], 0.0)
        o_ref[...] = jnp.dot(y, lw[...], preferred_element_type=_f32) + lb_[...]

    blk = pl.BlockSpec((_ROWS, LB), lambda i: (i, 0))
    in_specs = (
        [blk] * (2 * nin) +
        [pl.BlockSpec((512, 512), lambda i: (0, 0)),
         pl.BlockSpec((1, 512), lambda i: (0, 0)),
         pl.BlockSpec((512, 512), lambda i: (0, 0)),
         pl.BlockSpec((1, 512), lambda i: (0, 0)),
         pl.BlockSpec((512, 256), lambda i: (0, 0)),
         pl.BlockSpec((1, 256), lambda i: (0, 0))])
    return pl.pallas_call(
        body,
        grid=(N // _ROWS,),
        in_specs=in_specs,
        out_specs=pl.BlockSpec((_ROWS, 256), lambda i: (i, 0)),
        out_shape=jax.ShapeDtypeStruct((N, 256), _f32),
    )


def kernel(x, edge_index, W1_0, b1_0, W2_0, b2_0, W1_1, b1_1, W2_1, b2_1,
           W1_2, b1_2, W2_2, b2_2, lin_W, lin_b):
    src = edge_index[0]
    dst = edge_index[1]
    pad = E_PAD - E
    src = jnp.concatenate([src, jnp.zeros((pad,), jnp.int32)])
    dst = jnp.concatenate([dst, jnp.full((pad,), N, jnp.int32)])
    zeros_rows = jnp.zeros((RPT, LB), _f32)

    nb_in = 256 // LB
    nb_hid = 512 // LB
    seg_in = _make_seg_sum(nb_in)
    seg_hid = _make_seg_sum(nb_hid)
    mlp_in = _mlp_layer(nb_in)
    mlp_hid = _mlp_layer(nb_hid)
    mlpf = _mlp_final()

    h = [x[:, LB * i:LB * (i + 1)] for i in range(nb_in)]
    agg = seg_in(src, dst, zeros_rows, *h)
    h = mlp_in(*h, *agg, W1_0, b1_0.reshape(1, -1), W2_0, b2_0.reshape(1, -1))
    agg = seg_hid(src, dst, zeros_rows, *h)
    h = mlp_hid(*h, *agg, W1_1, b1_1.reshape(1, -1), W2_1, b2_1.reshape(1, -1))
    agg = seg_hid(src, dst, zeros_rows, *h)
    return mlpf(*h, *agg, W1_2, b1_2.reshape(1, -1), W2_2, b2_2.reshape(1, -1),
                lin_W, lin_b.reshape(1, -1))
